# baseline (speedup 1.0000x reference)
import functools

import jax
import jax.numpy as jnp
from jax import lax
from jax.experimental import pallas as pl
from jax.experimental.pallas import tpu as pltpu
from jax.experimental.pallas import tpu_sc as plsc

B = 16384

try:
    _info = plsc.get_sparse_core_info()
    NC, NS, L = _info.num_cores, _info.num_subcores, _info.num_lanes
except Exception:
    NC, NS, L = 2, 16, 16
NW = NC * NS
CPW = B // NW
HALF = CPW // 2
NCHUNK_H = HALF // L

_ERR = 0.09
_THR2 = 0.03


def _ranks(v):
    one = jnp.ones(v[0].shape, jnp.int32)
    zero = jnp.zeros(v[0].shape, jnp.int32)
    c = {}
    for a in range(4):
        for b in range(a + 1, 4):
            c[(a, b)] = jnp.where(v[a] <= v[b], one, zero)
    r0 = (one + one + one) - c[(0, 1)] - c[(0, 2)] - c[(0, 3)]
    r1 = (one + one) + c[(0, 1)] - c[(1, 2)] - c[(1, 3)]
    r2 = one + c[(0, 2)] + c[(1, 2)] - c[(2, 3)]
    r3 = c[(0, 3)] + c[(1, 3)] + c[(2, 3)]
    return [r0, r1, r2, r3]


def _group_body(p, t):
    shape = p[0].shape
    m = jnp.maximum(jnp.maximum(p[0], p[1]), jnp.maximum(p[2], p[3]))
    u = [jnp.exp(pk - m) for pk in p]
    s = u[0] + u[1] + u[2] + u[3]
    r3s = jnp.float32(3.0) / s
    o = [uk * r3s for uk in u]

    close4 = ((jnp.abs(o[0] - t[0]) <= _ERR) & (jnp.abs(o[1] - t[1]) <= _ERR)
              & (jnp.abs(o[2] - t[2]) <= _ERR) & (jnp.abs(o[3] - t[3]) <= _ERR))

    a0 = jnp.minimum(t[0], t[1]); a1 = jnp.maximum(t[0], t[1])
    b2 = jnp.minimum(t[2], t[3]); b3 = jnp.maximum(t[2], t[3])
    s0 = jnp.minimum(a0, b2); c2 = jnp.maximum(a0, b2)
    c1 = jnp.minimum(a1, b3)
    s1 = jnp.minimum(c1, c2); s2 = jnp.maximum(c1, c2)
    diff1 = jnp.abs(s0 - s1)
    diff2 = jnp.abs(s2 - s1)

    rt = _ranks(t)
    ro = _ranks(o)
    e = [ro[i] == rt[i] for i in range(4)]

    lt1 = diff1 < _THR2
    lt2 = diff2 < _THR2
    jump = close4 & ((lt1 & lt2) | (lt1 & e[2]) | (lt2 & e[0]))

    ione = jnp.ones(shape, jnp.int32)
    izero = jnp.zeros(shape, jnp.int32)

    iz = [jnp.where(t[k] == 0.0, ione, izero) for k in range(4)]
    judge0 = iz[0] + iz[1] + iz[2] + iz[3]
    eq_all = e[0] & e[2] & e[3]

    cnt = izero
    for i in range(4):
        cnt = cnt + jnp.where(e[i] & (ro[i] >= 2), ione, izero)
    cond_j2 = cnt == 2
    cond_j3 = ro[3] == 3

    one = jnp.full(shape, 1.0, jnp.float32)
    zero = jnp.full(shape, 0.0, jnp.float32)
    branch23 = jnp.where(judge0 == 2, jnp.where(cond_j2, one, zero),
                         jnp.where(judge0 == 3, jnp.where(cond_j3, one, zero), zero))
    j_lt2 = judge0 < 2
    calc = jnp.where(jump | j_lt2, one, branch23)
    acc = jnp.where(jump, one,
                    jnp.where(j_lt2, jnp.where(eq_all, one, zero), branch23))
    return calc, acc


def _contrib_valid(calc_num, acc_sum):
    one = jnp.full(calc_num.shape, 1.0, jnp.float32)
    zero = jnp.full(calc_num.shape, 0.0, jnp.float32)
    nz = calc_num != zero
    contrib = jnp.where(nz, acc_sum / jnp.maximum(calc_num, one), zero)
    valid = jnp.where(nz, one, zero)
    return contrib, valid


def _make_kernel():
    mesh = plsc.VectorSubcoreMesh(core_axis_name="c", subcore_axis_name="s")

    @functools.partial(
        pl.kernel,
        mesh=mesh,
        out_type=jax.ShapeDtypeStruct((NW, 2, L), jnp.float32),
        scratch_types=[
            pltpu.VMEM((48, CPW), jnp.float32),
            pltpu.VMEM((2, L), jnp.float32),
            pltpu.SemaphoreType.DMA,
            pltpu.SemaphoreType.DMA,
        ],
    )
    def recall_sc(x_hbm, out_hbm, x_v, acc_v, sem_a, sem_b):
        wid = lax.axis_index("s") * NC + lax.axis_index("c")
        cp_a = pltpu.async_copy(x_hbm.at[wid, :, pl.ds(0, HALF)],
                                x_v.at[:, pl.ds(0, HALF)], sem_a)
        cp_b = pltpu.async_copy(x_hbm.at[wid, :, pl.ds(HALF, HALF)],
                                x_v.at[:, pl.ds(HALF, HALF)], sem_b)

        def chunk(ci, carry):
            contrib_acc, valid_acc = carry
            base = pl.multiple_of(ci * L, L)
            calc_num = jnp.zeros((L,), jnp.float32)
            acc_sum = jnp.zeros((L,), jnp.float32)
            for g in range(6):
                p = [x_v[4 * g + k, pl.ds(base, L)] for k in range(4)]
                t = [x_v[24 + 4 * g + k, pl.ds(base, L)] for k in range(4)]
                calc, acc = _group_body(p, t)
                calc_num = calc_num + calc
                acc_sum = acc_sum + acc
            contrib, valid = _contrib_valid(calc_num, acc_sum)
            return contrib_acc + contrib, valid_acc + valid

        def chunk2(cj, carry):
            return chunk(2 * cj + 1, chunk(2 * cj, carry))

        zeros = (jnp.zeros((L,), jnp.float32), jnp.zeros((L,), jnp.float32))
        cp_a.wait()
        mid = lax.fori_loop(0, NCHUNK_H // 2, chunk2, zeros)
        cp_b.wait()
        contrib_acc, valid_acc = lax.fori_loop(
            NCHUNK_H // 2, NCHUNK_H, chunk2, mid)
        acc_v[0, :] = contrib_acc
        acc_v[1, :] = valid_acc
        pltpu.sync_copy(acc_v, out_hbm.at[wid])

    return recall_sc


_recall_sc_cache = []


def kernel(pre, tar):
    if not _recall_sc_cache:
        _recall_sc_cache.append(_make_kernel())
    x = jnp.concatenate([pre[:, :24], tar[:, :24]], axis=1)
    x_r = x.T.reshape(48, NW, CPW).transpose(1, 0, 2)
    parts = _recall_sc_cache[0](x_r)
    contrib_total = jnp.sum(parts[:, 0, :])
    valid_total = jnp.sum(parts[:, 1, :])
    return jnp.where(valid_total == 0.0, jnp.float32(0.0),
                     contrib_total / jnp.maximum(valid_total, 1.0))


# speedup vs baseline: 1.0430x; 1.0430x over previous
import functools

import jax
import jax.numpy as jnp
from jax import lax
from jax.experimental import pallas as pl
from jax.experimental.pallas import tpu as pltpu
from jax.experimental.pallas import tpu_sc as plsc

B = 16384

try:
    _info = plsc.get_sparse_core_info()
    NC, NS, L = _info.num_cores, _info.num_subcores, _info.num_lanes
except Exception:
    NC, NS, L = 2, 16, 16
NW = NC * NS
CPW = B // NW
HALF = CPW // 2
NCHUNK_H = HALF // L

_ERR = 0.09
_THR2 = 0.03


def _ranks(v):
    one = jnp.ones(v[0].shape, jnp.int32)
    zero = jnp.zeros(v[0].shape, jnp.int32)
    c = {}
    for a in range(4):
        for b in range(a + 1, 4):
            c[(a, b)] = jnp.where(v[a] <= v[b], one, zero)
    r0 = (one + one + one) - c[(0, 1)] - c[(0, 2)] - c[(0, 3)]
    r1 = (one + one) + c[(0, 1)] - c[(1, 2)] - c[(1, 3)]
    r2 = one + c[(0, 2)] + c[(1, 2)] - c[(2, 3)]
    r3 = c[(0, 3)] + c[(1, 3)] + c[(2, 3)]
    return [r0, r1, r2, r3]


def _group_body(p, t):
    shape = p[0].shape
    m = jnp.maximum(jnp.maximum(p[0], p[1]), jnp.maximum(p[2], p[3]))
    u = [jnp.exp(pk - m) for pk in p]
    s = u[0] + u[1] + u[2] + u[3]
    r3s = jnp.float32(3.0) / s
    o = [uk * r3s for uk in u]

    close4 = ((jnp.abs(o[0] - t[0]) <= _ERR) & (jnp.abs(o[1] - t[1]) <= _ERR)
              & (jnp.abs(o[2] - t[2]) <= _ERR) & (jnp.abs(o[3] - t[3]) <= _ERR))

    a0 = jnp.minimum(t[0], t[1]); a1 = jnp.maximum(t[0], t[1])
    b2 = jnp.minimum(t[2], t[3]); b3 = jnp.maximum(t[2], t[3])
    s0 = jnp.minimum(a0, b2); c2 = jnp.maximum(a0, b2)
    c1 = jnp.minimum(a1, b3)
    s1 = jnp.minimum(c1, c2); s2 = jnp.maximum(c1, c2)
    diff1 = jnp.abs(s0 - s1)
    diff2 = jnp.abs(s2 - s1)

    rt = _ranks(t)
    ro = _ranks(o)
    e = [ro[i] == rt[i] for i in range(4)]

    lt1 = diff1 < _THR2
    lt2 = diff2 < _THR2
    jump = close4 & ((lt1 & lt2) | (lt1 & e[2]) | (lt2 & e[0]))

    ione = jnp.ones(shape, jnp.int32)
    izero = jnp.zeros(shape, jnp.int32)

    iz = [jnp.where(t[k] == 0.0, ione, izero) for k in range(4)]
    judge0 = iz[0] + iz[1] + iz[2] + iz[3]
    eq_all = e[0] & e[2] & e[3]

    cnt = izero
    for i in range(4):
        cnt = cnt + jnp.where(e[i] & (ro[i] >= 2), ione, izero)
    cond_j2 = cnt == 2
    cond_j3 = ro[3] == 3

    one = jnp.full(shape, 1.0, jnp.float32)
    zero = jnp.full(shape, 0.0, jnp.float32)
    branch23 = jnp.where(judge0 == 2, jnp.where(cond_j2, one, zero),
                         jnp.where(judge0 == 3, jnp.where(cond_j3, one, zero), zero))
    j_lt2 = judge0 < 2
    calc = jnp.where(jump | j_lt2, one, branch23)
    acc = jnp.where(jump, one,
                    jnp.where(j_lt2, jnp.where(eq_all, one, zero), branch23))
    return calc, acc


def _contrib_valid(calc_num, acc_sum):
    one = jnp.full(calc_num.shape, 1.0, jnp.float32)
    zero = jnp.full(calc_num.shape, 0.0, jnp.float32)
    nz = calc_num != zero
    contrib = jnp.where(nz, acc_sum / jnp.maximum(calc_num, one), zero)
    valid = jnp.where(nz, one, zero)
    return contrib, valid


def _make_kernel():
    mesh = plsc.VectorSubcoreMesh(core_axis_name="c", subcore_axis_name="s")

    @functools.partial(
        pl.kernel,
        mesh=mesh,
        out_type=jax.ShapeDtypeStruct((NW, 2, L), jnp.float32),
        scratch_types=[
            pltpu.VMEM((48, CPW), jnp.float32),
            pltpu.VMEM((2, L), jnp.float32),
            pltpu.SemaphoreType.DMA,
            pltpu.SemaphoreType.DMA,
        ],
    )
    def recall_sc(x_hbm, out_hbm, x_v, acc_v, sem_a, sem_b):
        wid = lax.axis_index("s") * NC + lax.axis_index("c")
        cp_a = pltpu.async_copy(x_hbm.at[wid, :, pl.ds(0, HALF)],
                                x_v.at[:, pl.ds(0, HALF)], sem_a)
        cp_b = pltpu.async_copy(x_hbm.at[wid, :, pl.ds(HALF, HALF)],
                                x_v.at[:, pl.ds(HALF, HALF)], sem_b)

        def chunk(ci, carry):
            contrib_acc, valid_acc = carry
            base = pl.multiple_of(ci * L, L)
            calc_num = jnp.zeros((L,), jnp.float32)
            acc_sum = jnp.zeros((L,), jnp.float32)
            for g in range(6):
                p = [x_v[4 * g + k, pl.ds(base, L)] for k in range(4)]
                t = [x_v[24 + 4 * g + k, pl.ds(base, L)] for k in range(4)]
                calc, acc = _group_body(p, t)
                calc_num = calc_num + calc
                acc_sum = acc_sum + acc
            contrib, valid = _contrib_valid(calc_num, acc_sum)
            return contrib_acc + contrib, valid_acc + valid

        zeros = (jnp.zeros((L,), jnp.float32), jnp.zeros((L,), jnp.float32))
        cp_a.wait()
        mid = lax.fori_loop(0, NCHUNK_H, chunk, zeros)
        cp_b.wait()
        contrib_acc, valid_acc = lax.fori_loop(NCHUNK_H, 2 * NCHUNK_H, chunk, mid)
        acc_v[0, :] = contrib_acc
        acc_v[1, :] = valid_acc
        pltpu.sync_copy(acc_v, out_hbm.at[wid])

    return recall_sc


_recall_sc_cache = []


def kernel(pre, tar):
    if not _recall_sc_cache:
        _recall_sc_cache.append(_make_kernel())
    x = jnp.concatenate([pre[:, :24], tar[:, :24]], axis=1)
    x_r = x.T.reshape(48, NW, CPW).transpose(1, 0, 2)
    parts = _recall_sc_cache[0](x_r)
    contrib_total = jnp.sum(parts[:, 0, :])
    valid_total = jnp.sum(parts[:, 1, :])
    return jnp.where(valid_total == 0.0, jnp.float32(0.0),
                     contrib_total / jnp.maximum(valid_total, 1.0))


# speedup vs baseline: 1.0451x; 1.0020x over previous
import functools

import jax
import jax.numpy as jnp
from jax import lax
from jax.experimental import pallas as pl
from jax.experimental.pallas import tpu as pltpu
from jax.experimental.pallas import tpu_sc as plsc

B = 16384

try:
    _info = plsc.get_sparse_core_info()
    NC, NS, L = _info.num_cores, _info.num_subcores, _info.num_lanes
except Exception:
    NC, NS, L = 2, 16, 16
NW = NC * NS
CPW = B // NW
HALF = CPW // 2
NCHUNK_H = HALF // L

_ERR = 0.09
_THR2 = 0.03


def _ranks(v):
    one = jnp.ones(v[0].shape, jnp.int32)
    zero = jnp.zeros(v[0].shape, jnp.int32)
    c = {}
    for a in range(4):
        for b in range(a + 1, 4):
            c[(a, b)] = jnp.where(v[a] <= v[b], one, zero)
    r0 = (one + one + one) - c[(0, 1)] - c[(0, 2)] - c[(0, 3)]
    r1 = (one + one) + c[(0, 1)] - c[(1, 2)] - c[(1, 3)]
    r2 = one + c[(0, 2)] + c[(1, 2)] - c[(2, 3)]
    r3 = c[(0, 3)] + c[(1, 3)] + c[(2, 3)]
    return [r0, r1, r2, r3]


def _group_body(p, t):
    shape = p[0].shape
    m = jnp.maximum(jnp.maximum(p[0], p[1]), jnp.maximum(p[2], p[3]))
    u = [jnp.exp(pk - m) for pk in p]
    s = u[0] + u[1] + u[2] + u[3]
    r3s = jnp.float32(3.0) / s
    o = [uk * r3s for uk in u]

    close4 = ((jnp.abs(o[0] - t[0]) <= _ERR) & (jnp.abs(o[1] - t[1]) <= _ERR)
              & (jnp.abs(o[2] - t[2]) <= _ERR) & (jnp.abs(o[3] - t[3]) <= _ERR))

    a0 = jnp.minimum(t[0], t[1]); a1 = jnp.maximum(t[0], t[1])
    b2 = jnp.minimum(t[2], t[3]); b3 = jnp.maximum(t[2], t[3])
    s0 = jnp.minimum(a0, b2); c2 = jnp.maximum(a0, b2)
    c1 = jnp.minimum(a1, b3)
    s1 = jnp.minimum(c1, c2); s2 = jnp.maximum(c1, c2)
    diff1 = jnp.abs(s0 - s1)
    diff2 = jnp.abs(s2 - s1)

    rt = _ranks(t)
    ro = _ranks(o)
    e = [ro[i] == rt[i] for i in range(4)]

    lt1 = diff1 < _THR2
    lt2 = diff2 < _THR2
    jump = close4 & ((lt1 & lt2) | (lt1 & e[2]) | (lt2 & e[0]))

    ione = jnp.ones(shape, jnp.int32)
    izero = jnp.zeros(shape, jnp.int32)

    iz = [jnp.where(t[k] == 0.0, ione, izero) for k in range(4)]
    judge0 = iz[0] + iz[1] + iz[2] + iz[3]
    eq_all = e[0] & e[2] & e[3]

    cnt = izero
    for i in range(4):
        cnt = cnt + jnp.where(e[i] & (ro[i] >= 2), ione, izero)
    cond_j2 = cnt == 2
    cond_j3 = ro[3] == 3

    one = jnp.full(shape, 1.0, jnp.float32)
    zero = jnp.full(shape, 0.0, jnp.float32)
    branch23 = jnp.where(judge0 == 2, jnp.where(cond_j2, one, zero),
                         jnp.where(judge0 == 3, jnp.where(cond_j3, one, zero), zero))
    j_lt2 = judge0 < 2
    calc = jnp.where(jump | j_lt2, one, branch23)
    acc = jnp.where(jump, one,
                    jnp.where(j_lt2, jnp.where(eq_all, one, zero), branch23))
    return calc, acc


def _contrib_valid(calc_num, acc_sum):
    one = jnp.full(calc_num.shape, 1.0, jnp.float32)
    zero = jnp.full(calc_num.shape, 0.0, jnp.float32)
    nz = calc_num != zero
    contrib = jnp.where(nz, acc_sum / jnp.maximum(calc_num, one), zero)
    valid = jnp.where(nz, one, zero)
    return contrib, valid


def _make_kernel():
    mesh = plsc.VectorSubcoreMesh(core_axis_name="c", subcore_axis_name="s")

    @functools.partial(
        pl.kernel,
        mesh=mesh,
        out_type=jax.ShapeDtypeStruct((NW, 2, L), jnp.float32),
        scratch_types=[
            pltpu.VMEM((48, CPW), jnp.float32),
            pltpu.VMEM((2, L), jnp.float32),
            pltpu.SemaphoreType.DMA,
            pltpu.SemaphoreType.DMA,
        ],
    )
    def recall_sc(x_hbm, out_hbm, x_v, acc_v, sem_a, sem_b):
        wid = lax.axis_index("s") * NC + lax.axis_index("c")
        cp_a = pltpu.async_copy(x_hbm.at[wid, :, pl.ds(0, HALF)],
                                x_v.at[:, pl.ds(0, HALF)], sem_a)
        cp_b = pltpu.async_copy(x_hbm.at[wid, :, pl.ds(HALF, HALF)],
                                x_v.at[:, pl.ds(HALF, HALF)], sem_b)

        def chunk(ci, carry):
            contrib_acc, valid_acc = carry
            base = pl.multiple_of(ci * L, L)
            calc_num = jnp.zeros((L,), jnp.float32)
            acc_sum = jnp.zeros((L,), jnp.float32)
            for g in range(6):
                p = [x_v[4 * g + k, pl.ds(base, L)] for k in range(4)]
                t = [x_v[24 + 4 * g + k, pl.ds(base, L)] for k in range(4)]
                calc, acc = _group_body(p, t)
                calc_num = calc_num + calc
                acc_sum = acc_sum + acc
            contrib, valid = _contrib_valid(calc_num, acc_sum)
            return contrib_acc + contrib, valid_acc + valid

        zeros = (jnp.zeros((L,), jnp.float32), jnp.zeros((L,), jnp.float32))
        cp_a.wait()

        mid = plsc.parallel_loop(0, NCHUNK_H, carry=zeros)(chunk)
        cp_b.wait()
        contrib_acc, valid_acc = plsc.parallel_loop(
            NCHUNK_H, 2 * NCHUNK_H, carry=mid)(chunk)
        acc_v[0, :] = contrib_acc
        acc_v[1, :] = valid_acc
        pltpu.sync_copy(acc_v, out_hbm.at[wid])

    return recall_sc


_recall_sc_cache = []


def kernel(pre, tar):
    if not _recall_sc_cache:
        _recall_sc_cache.append(_make_kernel())
    x = jnp.concatenate([pre[:, :24], tar[:, :24]], axis=1)
    x_r = x.T.reshape(48, NW, CPW).transpose(1, 0, 2)
    parts = _recall_sc_cache[0](x_r)
    contrib_total = jnp.sum(parts[:, 0, :])
    valid_total = jnp.sum(parts[:, 1, :])
    return jnp.where(valid_total == 0.0, jnp.float32(0.0),
                     contrib_total / jnp.maximum(valid_total, 1.0))
